# baseline (speedup 1.0000x reference)
import functools

import jax
import jax.numpy as jnp
from jax import lax
from jax.experimental import pallas as pl
from jax.experimental.pallas import tpu as pltpu
from jax.experimental.pallas import tpu_sc as plsc

B = 2
N = 10000
E = 320000
H = 128

K = 125
TILES = 16
EPT = E // TILES
CHUNKS = EPT // K
ZROWS = 80
ZCH = N // ZROWS
ZITER = -(-ZCH // TILES)
SUPER = 16
NSUPER = CHUNKS // SUPER


def _linear_relu(x, W, b2d, bm):
    M = x.shape[0]
    half = (M // bm) // 2

    def body(x_ref, w_ref, b_ref, o0_ref, o1_ref):
        i = pl.program_id(0)
        acc = jnp.maximum(
            jnp.dot(x_ref[...], w_ref[...],
                    preferred_element_type=jnp.float32) + b_ref[...], 0.0)

        @pl.when(i < half)
        def _():
            o0_ref[...] = acc

        @pl.when(i >= half)
        def _():
            o1_ref[...] = acc

    return pl.pallas_call(
        body,
        grid=(M // bm,),
        in_specs=[
            pl.BlockSpec((bm, H), lambda i: (i, 0)),
            pl.BlockSpec((H, H), lambda i: (0, 0)),
            pl.BlockSpec((1, H), lambda i: (0, 0)),
        ],
        out_specs=[
            pl.BlockSpec((bm, H), lambda i: (jnp.minimum(i, half - 1), 0)),
            pl.BlockSpec((bm, H), lambda i: (jnp.maximum(i - half, 0), 0)),
        ],
        out_shape=[jax.ShapeDtypeStruct((N, H), jnp.float32),
                   jax.ShapeDtypeStruct((N, H), jnp.float32)],
    )(x, W, b2d)


def _final_head(aggr2, W2, b2d, Wc1, bc1_2d, Wc2, bc2_2d, bm):
    nblocks = (B * N) // bm
    per_batch = N // bm

    def body(a_ref, w2_ref, b2_ref, wc1_ref, bc1_ref, wc2_ref, bc2_ref,
             o_ref, acc_ref):
        i = pl.program_id(0)

        @pl.when(i == 0)
        def _():
            acc_ref[...] = jnp.zeros_like(acc_ref)

        h2 = jnp.maximum(
            jnp.dot(a_ref[...], w2_ref[...],
                    preferred_element_type=jnp.float32) + b2_ref[...], 0.0)
        colsum = jnp.sum(h2, axis=0, keepdims=True)

        @pl.when(i < per_batch)
        def _():
            acc_ref[0:1, :] += colsum

        @pl.when(i >= per_batch)
        def _():
            acc_ref[1:2, :] += colsum

        @pl.when(i == nblocks - 1)
        def _():
            hm = acc_ref[...] / float(N)
            z = jnp.maximum(
                jnp.dot(hm, wc1_ref[...],
                        preferred_element_type=jnp.float32) + bc1_ref[...],
                0.0)
            o_ref[...] = (jnp.dot(z, wc2_ref[...],
                                  preferred_element_type=jnp.float32)
                          + bc2_ref[...])

    return pl.pallas_call(
        body,
        grid=(nblocks,),
        in_specs=[
            pl.BlockSpec((bm, H), lambda i: (i, 0)),
            pl.BlockSpec((H, H), lambda i: (0, 0)),
            pl.BlockSpec((1, H), lambda i: (0, 0)),
            pl.BlockSpec((H, H // 2), lambda i: (0, 0)),
            pl.BlockSpec((1, H // 2), lambda i: (0, 0)),
            pl.BlockSpec((H // 2, 1), lambda i: (0, 0)),
            pl.BlockSpec((1, 1), lambda i: (0, 0)),
        ],
        out_specs=pl.BlockSpec((B, 1), lambda i: (0, 0)),
        out_shape=jax.ShapeDtypeStruct((B, 1), jnp.float32),
        scratch_shapes=[pltpu.VMEM((B, H), jnp.float32)],
    )(aggr2, W2, b2d, Wc1, bc1_2d, Wc2, bc2_2d)


def kernel(x, edge_index, W_embed, b_embed, W1, b1, W2, b2, Wc1, bc1, Wc2, bc2):
    x2 = x.reshape(B * N, H)
    rc = edge_index.astype(jnp.int32).reshape(2, E // K, K)
    zeros = jnp.zeros((ZROWS, H), jnp.float32)

    h0, h1 = _linear_relu(x2, W_embed, b_embed.reshape(1, H), bm=1000)

    aggr1 = _sc_aggregate_2core(h0, h1, rc, zeros)
    g0, g1 = _linear_relu(aggr1, W1, b1.reshape(1, H), bm=1000)
    aggr2 = _sc_aggregate_2core(g0, g1, rc, zeros)

    out = _final_head(aggr2, W2, b2.reshape(1, H),
                      Wc1, bc1.reshape(1, H // 2),
                      Wc2, bc2.reshape(1, 1), bm=1000)
    return out.reshape(B)


def _sc_aggregate_2core(h0, h1, rc, zeros):
    mesh = plsc.VectorSubcoreMesh(core_axis_name="c", subcore_axis_name="s",
                                  num_cores=2, num_subcores=TILES)

    @functools.partial(
        pl.kernel,
        out_type=jax.ShapeDtypeStruct((B * N, H), jnp.float32),
        mesh=mesh,
        scratch_types=[
            pltpu.VMEM_SHARED((N, H), jnp.float32),
            pltpu.VMEM((SUPER, K), jnp.int32),
            pltpu.VMEM((SUPER, K), jnp.int32),
            pltpu.VMEM((K, H), jnp.float32),
            pltpu.VMEM((K, H), jnp.float32),
            pltpu.SemaphoreType.DMA,
            pltpu.SemaphoreType.DMA,
        ],
    )
    def agg(h0_hbm, h1_hbm, rc_hbm, zeros_hbm, out_hbm,
            accum, ridx, cidx, buf0, buf1, sg0, sg1):
        c = lax.axis_index("c")
        s = lax.axis_index("s")
        zb = buf0.at[pl.ds(0, ZROWS)]

        pltpu.sync_copy(zeros_hbm, zb)
        for z in range(ZITER):
            cid = s + TILES * z

            @pl.when(cid < ZCH)
            def _():
                pltpu.sync_copy(zb, accum.at[pl.ds(cid * ZROWS, ZROWS)])

        plsc.subcore_barrier()

        def make_super_body(h_hbm):
            def super_body(g, _):
                base = s * CHUNKS + g * SUPER
                pltpu.sync_copy(rc_hbm.at[0].at[pl.ds(base, SUPER)], ridx)
                pltpu.sync_copy(rc_hbm.at[1].at[pl.ds(base, SUPER)], cidx)

                pltpu.async_copy(h_hbm.at[cidx.at[0]], buf0, sg0)

                def pair_body(p, _):
                    j0 = 2 * p
                    j1 = j0 + 1
                    pltpu.async_copy(h_hbm.at[cidx.at[j1]], buf1, sg1)
                    pltpu.make_async_copy(
                        h_hbm.at[cidx.at[j0]], buf0, sg0).wait()
                    pltpu.sync_copy(buf0, accum.at[ridx.at[j0]], add=True)

                    @pl.when(j1 + 1 < SUPER)
                    def _():
                        pltpu.async_copy(h_hbm.at[cidx.at[j1 + 1]], buf0, sg0)

                    pltpu.make_async_copy(
                        h_hbm.at[cidx.at[j1]], buf1, sg1).wait()
                    pltpu.sync_copy(buf1, accum.at[ridx.at[j1]], add=True)
                    return 0

                lax.fori_loop(0, SUPER // 2, pair_body, 0)
                return 0

            return super_body

        @pl.when(c == 0)
        def _():
            lax.fori_loop(0, NSUPER, make_super_body(h0_hbm), 0)

        @pl.when(c == 1)
        def _():
            lax.fori_loop(0, NSUPER, make_super_body(h1_hbm), 0)
        plsc.subcore_barrier()

        for z in range(ZITER):
            cid = s + TILES * z

            @pl.when(cid < ZCH)
            def _():
                pltpu.sync_copy(accum.at[pl.ds(cid * ZROWS, ZROWS)], zb)
                pltpu.sync_copy(
                    zb, out_hbm.at[pl.ds(c * N + cid * ZROWS, ZROWS)])

    return agg(h0, h1, rc, zeros)


# speedup vs baseline: 1.0433x; 1.0433x over previous
import functools

import jax
import jax.numpy as jnp
from jax import lax
from jax.experimental import pallas as pl
from jax.experimental.pallas import tpu as pltpu
from jax.experimental.pallas import tpu_sc as plsc

B = 2
N = 10000
E = 320000
H = 128

K = 125
TILES = 16
EPT = E // TILES
CHUNKS = EPT // K
ZROWS = 80
ZCH = N // ZROWS
ZITER = -(-ZCH // TILES)
SUPER = 16
NSUPER = CHUNKS // SUPER


def _linear_relu(x, W, b2d, bm):
    M = x.shape[0]
    half = (M // bm) // 2

    def body(x_ref, w_ref, b_ref, o0_ref, o1_ref):
        i = pl.program_id(0)
        acc = jnp.maximum(
            jnp.dot(x_ref[...], w_ref[...],
                    preferred_element_type=jnp.float32) + b_ref[...], 0.0)

        @pl.when(i < half)
        def _():
            o0_ref[...] = acc

        @pl.when(i >= half)
        def _():
            o1_ref[...] = acc

    return pl.pallas_call(
        body,
        grid=(M // bm,),
        in_specs=[
            pl.BlockSpec((bm, H), lambda i: (i, 0)),
            pl.BlockSpec((H, H), lambda i: (0, 0)),
            pl.BlockSpec((1, H), lambda i: (0, 0)),
        ],
        out_specs=[
            pl.BlockSpec((bm, H), lambda i: (jnp.minimum(i, half - 1), 0)),
            pl.BlockSpec((bm, H), lambda i: (jnp.maximum(i - half, 0), 0)),
        ],
        out_shape=[jax.ShapeDtypeStruct((N, H), jnp.float32),
                   jax.ShapeDtypeStruct((N, H), jnp.float32)],
    )(x, W, b2d)


def _final_head(aggr2, W2, b2d, Wc1, bc1_2d, Wc2, bc2_2d, bm):
    nblocks = (B * N) // bm
    per_batch = N // bm

    def body(a_ref, w2_ref, b2_ref, wc1_ref, bc1_ref, wc2_ref, bc2_ref,
             o_ref, acc_ref):
        i = pl.program_id(0)

        @pl.when(i == 0)
        def _():
            acc_ref[...] = jnp.zeros_like(acc_ref)

        h2 = jnp.maximum(
            jnp.dot(a_ref[...], w2_ref[...],
                    preferred_element_type=jnp.float32) + b2_ref[...], 0.0)
        colsum = jnp.sum(h2, axis=0, keepdims=True)

        @pl.when(i < per_batch)
        def _():
            acc_ref[0:1, :] += colsum

        @pl.when(i >= per_batch)
        def _():
            acc_ref[1:2, :] += colsum

        @pl.when(i == nblocks - 1)
        def _():
            hm = acc_ref[...] / float(N)
            z = jnp.maximum(
                jnp.dot(hm, wc1_ref[...],
                        preferred_element_type=jnp.float32) + bc1_ref[...],
                0.0)
            o_ref[...] = (jnp.dot(z, wc2_ref[...],
                                  preferred_element_type=jnp.float32)
                          + bc2_ref[...])

    return pl.pallas_call(
        body,
        grid=(nblocks,),
        in_specs=[
            pl.BlockSpec((bm, H), lambda i: (i, 0)),
            pl.BlockSpec((H, H), lambda i: (0, 0)),
            pl.BlockSpec((1, H), lambda i: (0, 0)),
            pl.BlockSpec((H, H // 2), lambda i: (0, 0)),
            pl.BlockSpec((1, H // 2), lambda i: (0, 0)),
            pl.BlockSpec((H // 2, 1), lambda i: (0, 0)),
            pl.BlockSpec((1, 1), lambda i: (0, 0)),
        ],
        out_specs=pl.BlockSpec((B, 1), lambda i: (0, 0)),
        out_shape=jax.ShapeDtypeStruct((B, 1), jnp.float32),
        scratch_shapes=[pltpu.VMEM((B, H), jnp.float32)],
    )(aggr2, W2, b2d, Wc1, bc1_2d, Wc2, bc2_2d)


def kernel(x, edge_index, W_embed, b_embed, W1, b1, W2, b2, Wc1, bc1, Wc2, bc2):
    x2 = x.reshape(B * N, H)
    rc = edge_index.astype(jnp.int32).reshape(2, E // K, K)
    zeros = jnp.zeros((ZROWS, H), jnp.float32)

    h0, h1 = _linear_relu(x2, W_embed, b_embed.reshape(1, H), bm=1000)

    aggr1 = _sc_aggregate_2core(h0, h1, rc, zeros)
    g0, g1 = _linear_relu(aggr1, W1, b1.reshape(1, H), bm=1000)
    aggr2 = _sc_aggregate_2core(g0, g1, rc, zeros)

    out = _final_head(aggr2, W2, b2.reshape(1, H),
                      Wc1, bc1.reshape(1, H // 2),
                      Wc2, bc2.reshape(1, 1), bm=1000)
    return out.reshape(B)


def _sc_aggregate_2core(h0, h1, rc, zeros):
    mesh = plsc.VectorSubcoreMesh(core_axis_name="c", subcore_axis_name="s",
                                  num_cores=2, num_subcores=TILES)

    @functools.partial(
        pl.kernel,
        out_type=jax.ShapeDtypeStruct((B * N, H), jnp.float32),
        mesh=mesh,
        scratch_types=[
            pltpu.VMEM_SHARED((N, H), jnp.float32),
            pltpu.VMEM((2, SUPER, K), jnp.int32),
            pltpu.VMEM((2, SUPER, K), jnp.int32),
            pltpu.VMEM((K, H), jnp.float32),
            pltpu.VMEM((K, H), jnp.float32),
            pltpu.SemaphoreType.DMA,
            pltpu.SemaphoreType.DMA,
            pltpu.SemaphoreType.DMA,
            pltpu.SemaphoreType.DMA,
        ],
    )
    def agg(h0_hbm, h1_hbm, rc_hbm, zeros_hbm, out_hbm,
            accum, ridx2, cidx2, buf0, buf1, sg0, sg1, si0, si1):
        c = lax.axis_index("c")
        s = lax.axis_index("s")
        zb = buf0.at[pl.ds(0, ZROWS)]

        pltpu.sync_copy(zeros_hbm, zb)
        for z in range(ZITER):
            cid = s + TILES * z

            @pl.when(cid < ZCH)
            def _():
                pltpu.sync_copy(zb, accum.at[pl.ds(cid * ZROWS, ZROWS)])

        plsc.subcore_barrier()

        def stage(g, slot_ridx, slot_cidx, sem):
            base = s * CHUNKS + g * SUPER
            pltpu.async_copy(rc_hbm.at[0].at[pl.ds(base, SUPER)],
                             slot_ridx, sem)
            pltpu.async_copy(rc_hbm.at[1].at[pl.ds(base, SUPER)],
                             slot_cidx, sem)

        def stage_wait(g, slot_ridx, slot_cidx, sem):
            base = s * CHUNKS + g * SUPER
            pltpu.make_async_copy(rc_hbm.at[0].at[pl.ds(base, SUPER)],
                                  slot_ridx, sem).wait()
            pltpu.make_async_copy(rc_hbm.at[1].at[pl.ds(base, SUPER)],
                                  slot_cidx, sem).wait()

        def make_block(h_hbm, ridx, cidx):
            def run_block():
                pltpu.async_copy(h_hbm.at[cidx.at[0]], buf0, sg0)

                def pair_body(p, _):
                    j0 = 2 * p
                    j1 = j0 + 1
                    pltpu.async_copy(h_hbm.at[cidx.at[j1]], buf1, sg1)
                    pltpu.make_async_copy(
                        h_hbm.at[cidx.at[j0]], buf0, sg0).wait()
                    pltpu.sync_copy(buf0, accum.at[ridx.at[j0]], add=True)

                    @pl.when(j1 + 1 < SUPER)
                    def _():
                        pltpu.async_copy(h_hbm.at[cidx.at[j1 + 1]], buf0, sg0)

                    pltpu.make_async_copy(
                        h_hbm.at[cidx.at[j1]], buf1, sg1).wait()
                    pltpu.sync_copy(buf1, accum.at[ridx.at[j1]], add=True)
                    return 0

                lax.fori_loop(0, SUPER // 2, pair_body, 0)

            return run_block

        def make_super_body(h_hbm):
            def super_body(q, _):
                g0 = 2 * q
                g1 = g0 + 1
                stage_wait(g0, ridx2.at[0], cidx2.at[0], si0)
                stage(g1, ridx2.at[1], cidx2.at[1], si1)
                make_block(h_hbm, ridx2.at[0], cidx2.at[0])()
                stage_wait(g1, ridx2.at[1], cidx2.at[1], si1)

                @pl.when(g1 + 1 < NSUPER)
                def _():
                    stage(g1 + 1, ridx2.at[0], cidx2.at[0], si0)

                make_block(h_hbm, ridx2.at[1], cidx2.at[1])()
                return 0

            return super_body

        stage(0, ridx2.at[0], cidx2.at[0], si0)

        @pl.when(c == 0)
        def _():
            lax.fori_loop(0, NSUPER // 2, make_super_body(h0_hbm), 0)

        @pl.when(c == 1)
        def _():
            lax.fori_loop(0, NSUPER // 2, make_super_body(h1_hbm), 0)
        plsc.subcore_barrier()

        for z in range(ZITER):
            cid = s + TILES * z

            @pl.when(cid < ZCH)
            def _():
                pltpu.sync_copy(accum.at[pl.ds(cid * ZROWS, ZROWS)], zb)
                pltpu.sync_copy(
                    zb, out_hbm.at[pl.ds(c * N + cid * ZROWS, ZROWS)])

    return agg(h0, h1, rc, zeros)


# speedup vs baseline: 1.0792x; 1.0344x over previous
import functools

import jax
import jax.numpy as jnp
from jax import lax
from jax.experimental import pallas as pl
from jax.experimental.pallas import tpu as pltpu
from jax.experimental.pallas import tpu_sc as plsc

B = 2
N = 10000
E = 320000
H = 128

K = 125
TILES = 16
EPT = E // TILES
CHUNKS = EPT // K
ZROWS = 80
ZCH = N // ZROWS
ZITER = -(-ZCH // TILES)
SUPER = 16
NSUPER = CHUNKS // SUPER


def _linear_relu(x, W, b2d, bm):
    M = x.shape[0]
    half = (M // bm) // 2

    def body(x_ref, w_ref, b_ref, o0_ref, o1_ref):
        i = pl.program_id(0)
        acc = jnp.maximum(
            jnp.dot(x_ref[...], w_ref[...],
                    preferred_element_type=jnp.float32) + b_ref[...], 0.0)

        @pl.when(i < half)
        def _():
            o0_ref[...] = acc

        @pl.when(i >= half)
        def _():
            o1_ref[...] = acc

    return pl.pallas_call(
        body,
        grid=(M // bm,),
        in_specs=[
            pl.BlockSpec((bm, H), lambda i: (i, 0)),
            pl.BlockSpec((H, H), lambda i: (0, 0)),
            pl.BlockSpec((1, H), lambda i: (0, 0)),
        ],
        out_specs=[
            pl.BlockSpec((bm, H), lambda i: (jnp.minimum(i, half - 1), 0)),
            pl.BlockSpec((bm, H), lambda i: (jnp.maximum(i - half, 0), 0)),
        ],
        out_shape=[jax.ShapeDtypeStruct((N, H), jnp.float32),
                   jax.ShapeDtypeStruct((N, H), jnp.float32)],
    )(x, W, b2d)


def _final_head(aggr2, W2, b2d, Wc1, bc1_2d, Wc2, bc2_2d, bm):
    nblocks = (B * N) // bm
    per_batch = N // bm

    def body(a_ref, w2_ref, b2_ref, wc1_ref, bc1_ref, wc2_ref, bc2_ref,
             o_ref, acc_ref):
        i = pl.program_id(0)

        @pl.when(i == 0)
        def _():
            acc_ref[...] = jnp.zeros_like(acc_ref)

        h2 = jnp.maximum(
            jnp.dot(a_ref[...], w2_ref[...],
                    preferred_element_type=jnp.float32) + b2_ref[...], 0.0)
        colsum = jnp.sum(h2, axis=0, keepdims=True)

        @pl.when(i < per_batch)
        def _():
            acc_ref[0:1, :] += colsum

        @pl.when(i >= per_batch)
        def _():
            acc_ref[1:2, :] += colsum

        @pl.when(i == nblocks - 1)
        def _():
            hm = acc_ref[...] / float(N)
            z = jnp.maximum(
                jnp.dot(hm, wc1_ref[...],
                        preferred_element_type=jnp.float32) + bc1_ref[...],
                0.0)
            o_ref[...] = (jnp.dot(z, wc2_ref[...],
                                  preferred_element_type=jnp.float32)
                          + bc2_ref[...])

    return pl.pallas_call(
        body,
        grid=(nblocks,),
        in_specs=[
            pl.BlockSpec((bm, H), lambda i: (i, 0)),
            pl.BlockSpec((H, H), lambda i: (0, 0)),
            pl.BlockSpec((1, H), lambda i: (0, 0)),
            pl.BlockSpec((H, H // 2), lambda i: (0, 0)),
            pl.BlockSpec((1, H // 2), lambda i: (0, 0)),
            pl.BlockSpec((H // 2, 1), lambda i: (0, 0)),
            pl.BlockSpec((1, 1), lambda i: (0, 0)),
        ],
        out_specs=pl.BlockSpec((B, 1), lambda i: (0, 0)),
        out_shape=jax.ShapeDtypeStruct((B, 1), jnp.float32),
        scratch_shapes=[pltpu.VMEM((B, H), jnp.float32)],
    )(aggr2, W2, b2d, Wc1, bc1_2d, Wc2, bc2_2d)


def kernel(x, edge_index, W_embed, b_embed, W1, b1, W2, b2, Wc1, bc1, Wc2, bc2):
    x2 = x.reshape(B * N, H)
    rc = edge_index.astype(jnp.int32).reshape(2, E // K, K)
    zeros = jnp.zeros((ZROWS, H), jnp.float32)

    h0, h1 = _linear_relu(x2, W_embed, b_embed.reshape(1, H), bm=2000)

    aggr1 = _sc_aggregate_2core(h0, h1, rc, zeros)
    g0, g1 = _linear_relu(aggr1, W1, b1.reshape(1, H), bm=2000)
    aggr2 = _sc_aggregate_2core(g0, g1, rc, zeros)

    out = _final_head(aggr2, W2, b2.reshape(1, H),
                      Wc1, bc1.reshape(1, H // 2),
                      Wc2, bc2.reshape(1, 1), bm=2000)
    return out.reshape(B)


def _sc_aggregate_2core(h0, h1, rc, zeros):
    mesh = plsc.VectorSubcoreMesh(core_axis_name="c", subcore_axis_name="s",
                                  num_cores=2, num_subcores=TILES)

    @functools.partial(
        pl.kernel,
        out_type=jax.ShapeDtypeStruct((B * N, H), jnp.float32),
        mesh=mesh,
        scratch_types=[
            pltpu.VMEM_SHARED((N, H), jnp.float32),
            pltpu.VMEM((2, SUPER, K), jnp.int32),
            pltpu.VMEM((2, SUPER, K), jnp.int32),
            pltpu.VMEM((K, H), jnp.float32),
            pltpu.VMEM((K, H), jnp.float32),
            pltpu.SemaphoreType.DMA,
            pltpu.SemaphoreType.DMA,
            pltpu.SemaphoreType.DMA,
            pltpu.SemaphoreType.DMA,
        ],
    )
    def agg(h0_hbm, h1_hbm, rc_hbm, zeros_hbm, out_hbm,
            accum, ridx2, cidx2, buf0, buf1, sg0, sg1, si0, si1):
        c = lax.axis_index("c")
        s = lax.axis_index("s")
        zb = buf0.at[pl.ds(0, ZROWS)]

        pltpu.sync_copy(zeros_hbm, zb)
        for z in range(ZITER):
            cid = s + TILES * z

            @pl.when(cid < ZCH)
            def _():
                pltpu.sync_copy(zb, accum.at[pl.ds(cid * ZROWS, ZROWS)])

        plsc.subcore_barrier()

        def stage(g, slot_ridx, slot_cidx, sem):
            base = s * CHUNKS + g * SUPER
            pltpu.async_copy(rc_hbm.at[0].at[pl.ds(base, SUPER)],
                             slot_ridx, sem)
            pltpu.async_copy(rc_hbm.at[1].at[pl.ds(base, SUPER)],
                             slot_cidx, sem)

        def stage_wait(g, slot_ridx, slot_cidx, sem):
            base = s * CHUNKS + g * SUPER
            pltpu.make_async_copy(rc_hbm.at[0].at[pl.ds(base, SUPER)],
                                  slot_ridx, sem).wait()
            pltpu.make_async_copy(rc_hbm.at[1].at[pl.ds(base, SUPER)],
                                  slot_cidx, sem).wait()

        def make_block(h_hbm, ridx, cidx):
            def run_block():
                pltpu.async_copy(h_hbm.at[cidx.at[0]], buf0, sg0)

                def pair_body(p, _):
                    j0 = 2 * p
                    j1 = j0 + 1
                    pltpu.async_copy(h_hbm.at[cidx.at[j1]], buf1, sg1)
                    pltpu.make_async_copy(
                        h_hbm.at[cidx.at[j0]], buf0, sg0).wait()
                    pltpu.sync_copy(buf0, accum.at[ridx.at[j0]], add=True)

                    @pl.when(j1 + 1 < SUPER)
                    def _():
                        pltpu.async_copy(h_hbm.at[cidx.at[j1 + 1]], buf0, sg0)

                    pltpu.make_async_copy(
                        h_hbm.at[cidx.at[j1]], buf1, sg1).wait()
                    pltpu.sync_copy(buf1, accum.at[ridx.at[j1]], add=True)
                    return 0

                lax.fori_loop(0, SUPER // 2, pair_body, 0)

            return run_block

        def make_super_body(h_hbm):
            def super_body(q, _):
                g0 = 2 * q
                g1 = g0 + 1
                stage_wait(g0, ridx2.at[0], cidx2.at[0], si0)
                stage(g1, ridx2.at[1], cidx2.at[1], si1)
                make_block(h_hbm, ridx2.at[0], cidx2.at[0])()
                stage_wait(g1, ridx2.at[1], cidx2.at[1], si1)

                @pl.when(g1 + 1 < NSUPER)
                def _():
                    stage(g1 + 1, ridx2.at[0], cidx2.at[0], si0)

                make_block(h_hbm, ridx2.at[1], cidx2.at[1])()
                return 0

            return super_body

        stage(0, ridx2.at[0], cidx2.at[0], si0)

        @pl.when(c == 0)
        def _():
            lax.fori_loop(0, NSUPER // 2, make_super_body(h0_hbm), 0)

        @pl.when(c == 1)
        def _():
            lax.fori_loop(0, NSUPER // 2, make_super_body(h1_hbm), 0)
        plsc.subcore_barrier()

        for z in range(ZITER):
            cid = s + TILES * z

            @pl.when(cid < ZCH)
            def _():
                pltpu.sync_copy(accum.at[pl.ds(cid * ZROWS, ZROWS)], zb)
                pltpu.sync_copy(
                    zb, out_hbm.at[pl.ds(c * N + cid * ZROWS, ZROWS)])

    return agg(h0, h1, rc, zeros)


# speedup vs baseline: 1.0912x; 1.0111x over previous
import functools

import jax
import jax.numpy as jnp
from jax import lax
from jax.experimental import pallas as pl
from jax.experimental.pallas import tpu as pltpu
from jax.experimental.pallas import tpu_sc as plsc

B = 2
N = 10000
E = 320000
H = 128

K = 125
TILES = 16
EPT = E // TILES
CHUNKS = EPT // K
ZROWS = 80
ZCH = N // ZROWS
ZITER = -(-ZCH // TILES)
SUPER = 16
NSUPER = CHUNKS // SUPER


def _linear_relu(x, W, b2d, bm):
    M = x.shape[0]
    half = (M // bm) // 2

    def body(x_ref, w_ref, b_ref, o0_ref, o1_ref):
        i = pl.program_id(0)
        acc = jnp.maximum(
            jnp.dot(x_ref[...], w_ref[...],
                    preferred_element_type=jnp.float32) + b_ref[...], 0.0)

        @pl.when(i < half)
        def _():
            o0_ref[...] = acc

        @pl.when(i >= half)
        def _():
            o1_ref[...] = acc

    return pl.pallas_call(
        body,
        grid=(M // bm,),
        in_specs=[
            pl.BlockSpec((bm, H), lambda i: (i, 0)),
            pl.BlockSpec((H, H), lambda i: (0, 0)),
            pl.BlockSpec((1, H), lambda i: (0, 0)),
        ],
        out_specs=[
            pl.BlockSpec((bm, H), lambda i: (jnp.minimum(i, half - 1), 0)),
            pl.BlockSpec((bm, H), lambda i: (jnp.maximum(i - half, 0), 0)),
        ],
        out_shape=[jax.ShapeDtypeStruct((N, H), jnp.float32),
                   jax.ShapeDtypeStruct((N, H), jnp.float32)],
    )(x, W, b2d)


def _final_head(aggr2, W2, b2d, Wc1, bc1_2d, Wc2, bc2_2d, bm):
    nblocks = (B * N) // bm
    per_batch = N // bm

    def body(a_ref, w2_ref, b2_ref, wc1_ref, bc1_ref, wc2_ref, bc2_ref,
             o_ref, acc_ref):
        i = pl.program_id(0)

        @pl.when(i == 0)
        def _():
            acc_ref[...] = jnp.zeros_like(acc_ref)

        h2 = jnp.maximum(
            jnp.dot(a_ref[...], w2_ref[...],
                    preferred_element_type=jnp.float32) + b2_ref[...], 0.0)
        colsum = jnp.sum(h2, axis=0, keepdims=True)

        @pl.when(i < per_batch)
        def _():
            acc_ref[0:1, :] += colsum

        @pl.when(i >= per_batch)
        def _():
            acc_ref[1:2, :] += colsum

        @pl.when(i == nblocks - 1)
        def _():
            hm = acc_ref[...] / float(N)
            z = jnp.maximum(
                jnp.dot(hm, wc1_ref[...],
                        preferred_element_type=jnp.float32) + bc1_ref[...],
                0.0)
            o_ref[...] = (jnp.dot(z, wc2_ref[...],
                                  preferred_element_type=jnp.float32)
                          + bc2_ref[...])

    return pl.pallas_call(
        body,
        grid=(nblocks,),
        in_specs=[
            pl.BlockSpec((bm, H), lambda i: (i, 0)),
            pl.BlockSpec((H, H), lambda i: (0, 0)),
            pl.BlockSpec((1, H), lambda i: (0, 0)),
            pl.BlockSpec((H, H // 2), lambda i: (0, 0)),
            pl.BlockSpec((1, H // 2), lambda i: (0, 0)),
            pl.BlockSpec((H // 2, 1), lambda i: (0, 0)),
            pl.BlockSpec((1, 1), lambda i: (0, 0)),
        ],
        out_specs=pl.BlockSpec((B, 1), lambda i: (0, 0)),
        out_shape=jax.ShapeDtypeStruct((B, 1), jnp.float32),
        scratch_shapes=[pltpu.VMEM((B, H), jnp.float32)],
    )(aggr2, W2, b2d, Wc1, bc1_2d, Wc2, bc2_2d)


def kernel(x, edge_index, W_embed, b_embed, W1, b1, W2, b2, Wc1, bc1, Wc2, bc2):
    x2 = x.reshape(B * N, H)
    rc = edge_index.astype(jnp.int32).reshape(2, E // K, K)
    zeros = jnp.zeros((ZROWS, H), jnp.float32)

    h0, h1 = _linear_relu(x2, W_embed, b_embed.reshape(1, H), bm=2000)

    aggr1 = _sc_aggregate_2core(h0, h1, rc, zeros)
    g0, g1 = _linear_relu(aggr1, W1, b1.reshape(1, H), bm=2000)
    aggr2 = _sc_aggregate_2core(g0, g1, rc, zeros)

    out = _final_head(aggr2, W2, b2.reshape(1, H),
                      Wc1, bc1.reshape(1, H // 2),
                      Wc2, bc2.reshape(1, 1), bm=2000)
    return out.reshape(B)


def _sc_aggregate_2core(h0, h1, rc, zeros):
    mesh = plsc.VectorSubcoreMesh(core_axis_name="c", subcore_axis_name="s",
                                  num_cores=2, num_subcores=TILES)

    @functools.partial(
        pl.kernel,
        out_type=jax.ShapeDtypeStruct((B * N, H), jnp.float32),
        mesh=mesh,
        scratch_types=[
            pltpu.VMEM_SHARED((N, H), jnp.float32),
            pltpu.VMEM((2, SUPER, K), jnp.int32),
            pltpu.VMEM((2, SUPER, K), jnp.int32),
            pltpu.VMEM((K, H), jnp.float32),
            pltpu.VMEM((K, H), jnp.float32),
            pltpu.SemaphoreType.DMA,
            pltpu.SemaphoreType.DMA,
            pltpu.SemaphoreType.DMA,
            pltpu.SemaphoreType.DMA,
            pltpu.SemaphoreType.DMA,
        ],
    )
    def agg(h0_hbm, h1_hbm, rc_hbm, zeros_hbm, out_hbm,
            accum, ridx2, cidx2, buf0, buf1, sg0, sg1, si0, si1, sz):
        c = lax.axis_index("c")
        s = lax.axis_index("s")
        zb = buf0.at[pl.ds(0, ZROWS)]

        pltpu.sync_copy(zeros_hbm, zb)
        for z in range(ZITER):
            cid = s + TILES * z

            @pl.when(cid < ZCH)
            def _():
                pltpu.async_copy(zb, accum.at[pl.ds(cid * ZROWS, ZROWS)], sz)

        for z in range(ZITER):
            cid = s + TILES * z

            @pl.when(cid < ZCH)
            def _():
                pltpu.make_async_copy(
                    zb, accum.at[pl.ds(cid * ZROWS, ZROWS)], sz).wait()

        plsc.subcore_barrier()

        def stage(g, slot_ridx, slot_cidx, sem):
            base = s * CHUNKS + g * SUPER
            pltpu.async_copy(rc_hbm.at[0].at[pl.ds(base, SUPER)],
                             slot_ridx, sem)
            pltpu.async_copy(rc_hbm.at[1].at[pl.ds(base, SUPER)],
                             slot_cidx, sem)

        def stage_wait(g, slot_ridx, slot_cidx, sem):
            base = s * CHUNKS + g * SUPER
            pltpu.make_async_copy(rc_hbm.at[0].at[pl.ds(base, SUPER)],
                                  slot_ridx, sem).wait()
            pltpu.make_async_copy(rc_hbm.at[1].at[pl.ds(base, SUPER)],
                                  slot_cidx, sem).wait()

        def make_block(h_hbm, ridx, cidx):
            def run_block():
                pltpu.async_copy(h_hbm.at[cidx.at[0]], buf0, sg0)

                def pair_body(p, _):
                    j0 = 2 * p
                    j1 = j0 + 1
                    pltpu.async_copy(h_hbm.at[cidx.at[j1]], buf1, sg1)
                    pltpu.make_async_copy(
                        h_hbm.at[cidx.at[j0]], buf0, sg0).wait()
                    pltpu.sync_copy(buf0, accum.at[ridx.at[j0]], add=True)

                    @pl.when(j1 + 1 < SUPER)
                    def _():
                        pltpu.async_copy(h_hbm.at[cidx.at[j1 + 1]], buf0, sg0)

                    pltpu.make_async_copy(
                        h_hbm.at[cidx.at[j1]], buf1, sg1).wait()
                    pltpu.sync_copy(buf1, accum.at[ridx.at[j1]], add=True)
                    return 0

                lax.fori_loop(0, SUPER // 2, pair_body, 0)

            return run_block

        def make_super_body(h_hbm):
            def super_body(q, _):
                g0 = 2 * q
                g1 = g0 + 1
                stage_wait(g0, ridx2.at[0], cidx2.at[0], si0)
                stage(g1, ridx2.at[1], cidx2.at[1], si1)
                make_block(h_hbm, ridx2.at[0], cidx2.at[0])()
                stage_wait(g1, ridx2.at[1], cidx2.at[1], si1)

                @pl.when(g1 + 1 < NSUPER)
                def _():
                    stage(g1 + 1, ridx2.at[0], cidx2.at[0], si0)

                make_block(h_hbm, ridx2.at[1], cidx2.at[1])()
                return 0

            return super_body

        stage(0, ridx2.at[0], cidx2.at[0], si0)

        @pl.when(c == 0)
        def _():
            lax.fori_loop(0, NSUPER // 2, make_super_body(h0_hbm), 0)

        @pl.when(c == 1)
        def _():
            lax.fori_loop(0, NSUPER // 2, make_super_body(h1_hbm), 0)
        plsc.subcore_barrier()

        rbuf = [buf0.at[pl.ds(0, ZROWS)], buf1.at[pl.ds(0, ZROWS)]]

        def rb_src(z):
            return accum.at[pl.ds((s + TILES * z) * ZROWS, ZROWS)]

        def rb_dst(z):
            return out_hbm.at[pl.ds(c * N + (s + TILES * z) * ZROWS, ZROWS)]

        for z in range(ZITER):
            cid = s + TILES * z

            @pl.when(cid < ZCH)
            def _():
                if z >= 2:
                    pltpu.make_async_copy(
                        rbuf[z % 2], rb_dst(z - 2), sz).wait()
                pltpu.sync_copy(rb_src(z), rbuf[z % 2])
                pltpu.async_copy(rbuf[z % 2], rb_dst(z), sz)

        for z in range(ZITER - 2, ZITER):
            cid = s + TILES * z

            @pl.when(cid < ZCH)
            def _():
                pltpu.make_async_copy(rbuf[z % 2], rb_dst(z), sz).wait()

    return agg(h0, h1, rc, zeros)
